# initial kernel scaffold (unmeasured)
import jax
import jax.numpy as jnp
from jax import lax
from jax.experimental import pallas as pl
from jax.experimental.pallas import tpu as pltpu

N_DEV = 4
SCALE = 0.08838834764831843
SQ = 256
SKV = 4096
HQ = 8
DH = 128
NB = 4
BQ = 64
GK = SKV // (NB * BQ)
KV_R = GK * BQ
D_MODEL = HQ * DH

PKT_ROWS = 2 * SQ + 4


def kernel(x, Wq, K_ext, V_ext, Wo):
    x2 = x.reshape(SQ, D_MODEL)
    K2 = K_ext.reshape(SKV, HQ, DH)
    V2 = V_ext.reshape(SKV, HQ, DH)

    def body(x_ref, wq_ref, k_ref, v_ref, wo_ref, out_ref,
             comm_ref, send_sems, recv_sems):
        my = lax.axis_index("i")
        left = (my + N_DEV - 1) % N_DEV
        right = (my + 1) % N_DEV

        barrier_sem = pltpu.get_barrier_semaphore()
        for nbr in (left, right):
            pl.semaphore_signal(barrier_sem, inc=1, device_id=(nbr,),
                                device_id_type=pl.DeviceIdType.MESH)
        pl.semaphore_wait(barrier_sem, 2)

        q = jnp.dot(x_ref[...], wq_ref[...],
                    preferred_element_type=jnp.float32) * SCALE
        qt = q.reshape(NB, BQ, HQ, DH).transpose(0, 2, 1, 3)

        kp = (k_ref[...].reshape(GK, NB, BQ, HQ, DH)
              .transpose(1, 3, 0, 2, 4).reshape(NB, HQ, KV_R, DH))
        vp = (v_ref[...].reshape(GK, NB, BQ, HQ, DH)
              .transpose(1, 3, 0, 2, 4).reshape(NB, HQ, KV_R, DH))

        def partial(qv):
            s = jnp.einsum('rhqd,rhkd->rhqk', qv, kp,
                           preferred_element_type=jnp.float32)
            m = jnp.max(s, axis=-1)
            w = jnp.exp(s - m[..., None])
            l = jnp.sum(w, axis=-1)
            o = jnp.einsum('rhqk,rhkd->rhqd', w, vp,
                           preferred_element_type=jnp.float32)
            return m, l, o

        def write_acc(slot, m, l, o):
            comm_ref[slot, SQ:2 * SQ, :] = o.reshape(SQ, D_MODEL)
            comm_ref[slot, 2 * SQ:2 * SQ + 2, :] = m.reshape(2, D_MODEL)
            comm_ref[slot, 2 * SQ + 2:2 * SQ + 4, :] = l.reshape(2, D_MODEL)

        m0, l0, o0 = partial(qt)
        comm_ref[0, 0:SQ, :] = qt.reshape(SQ, D_MODEL)
        write_acc(0, m0, l0, o0)

        for h in range(N_DEV):
            rdma = pltpu.make_async_remote_copy(
                src_ref=comm_ref.at[h],
                dst_ref=comm_ref.at[h + 1],
                send_sem=send_sems.at[h],
                recv_sem=recv_sems.at[h],
                device_id=(right,),
                device_id_type=pl.DeviceIdType.MESH,
            )
            rdma.start()
            rdma.wait()
            if h < N_DEV - 1:
                qv = comm_ref[h + 1, 0:SQ, :].reshape(NB, HQ, BQ, DH)
                o_in = comm_ref[h + 1, SQ:2 * SQ, :].reshape(NB, HQ, BQ, DH)
                m_in = comm_ref[h + 1, 2 * SQ:2 * SQ + 2, :].reshape(NB, HQ, BQ)
                l_in = comm_ref[h + 1, 2 * SQ + 2:2 * SQ + 4, :].reshape(NB, HQ, BQ)
                m2, l2, o2 = partial(qv)
                m_new = jnp.maximum(m_in, m2)
                a = jnp.exp(m_in - m_new)
                b = jnp.exp(m2 - m_new)
                write_acc(h + 1, m_new, l_in * a + l2 * b,
                          o_in * a[..., None] + o2 * b[..., None])

        o_f = comm_ref[N_DEV, SQ:2 * SQ, :].reshape(NB, HQ, BQ, DH)
        l_f = comm_ref[N_DEV, 2 * SQ + 2:2 * SQ + 4, :].reshape(NB, HQ, BQ)
        ctx = (o_f / l_f[..., None]).transpose(0, 2, 1, 3).reshape(SQ, D_MODEL)
        out_ref[...] = jnp.dot(ctx, wo_ref[...],
                               preferred_element_type=jnp.float32)

    out = pl.pallas_call(
        body,
        out_shape=jax.ShapeDtypeStruct((SQ, D_MODEL), jnp.float32),
        in_specs=[pl.BlockSpec(memory_space=pltpu.VMEM)] * 5,
        out_specs=pl.BlockSpec(memory_space=pltpu.VMEM),
        scratch_shapes=[
            pltpu.VMEM((N_DEV + 1, PKT_ROWS, D_MODEL), jnp.float32),
            pltpu.SemaphoreType.DMA((N_DEV,)),
            pltpu.SemaphoreType.DMA((N_DEV,)),
        ],
        compiler_params=pltpu.CompilerParams(collective_id=0),
    )(x2, Wq, K2, V2, Wo)
    return out.reshape(1, SQ, D_MODEL)


# baseline (device time: 180845 ns/iter reference)
import jax
import jax.numpy as jnp
from jax import lax
from jax.experimental import pallas as pl
from jax.experimental.pallas import tpu as pltpu

N_DEV = 4
SCALE = 0.08838834764831843
SQ = 256
SKV = 4096
HQ = 8
DH = 128
NB = 4
BQ = 64
GK = SKV // (NB * BQ)
KV_R = GK * BQ
D_MODEL = HQ * DH


def kernel(x, Wq, K_ext, V_ext, Wo):
    x2 = x.reshape(SQ, D_MODEL)
    K4 = K_ext.reshape(GK, NB, BQ, D_MODEL)
    V4 = V_ext.reshape(GK, NB, BQ, D_MODEL)

    def body(x_ref, wq_ref, k_ref, v_ref, wo_ref, out_ref,
             comm_ref, l_ref, kp_ref, vp_ref,
             send_sems, recv_sems, send_sems_l, recv_sems_l, kv_sems):
        my = lax.axis_index("i")
        left = (my + N_DEV - 1) % N_DEV
        right = (my + 1) % N_DEV

        kv_copies = []
        for r in range(NB):
            c = pltpu.make_async_copy(k_ref.at[:, r], kp_ref.at[r],
                                      kv_sems.at[r])
            c.start()
            kv_copies.append(c)
            c = pltpu.make_async_copy(v_ref.at[:, r], vp_ref.at[r],
                                      kv_sems.at[NB + r])
            c.start()
            kv_copies.append(c)

        barrier_sem = pltpu.get_barrier_semaphore()
        for nbr in (left, right):
            pl.semaphore_signal(barrier_sem, inc=1, device_id=(nbr,),
                                device_id_type=pl.DeviceIdType.MESH)
        pl.semaphore_wait(barrier_sem, 2)

        q = jnp.dot(x_ref[...], wq_ref[...],
                    preferred_element_type=jnp.float32) * SCALE

        for c in kv_copies:
            c.wait()

        def accumulate(slot, first):
            for r in range(NB):
                for h in range(HQ):
                    hc = slice(h * DH, (h + 1) * DH)
                    rr = slice(r * BQ, (r + 1) * BQ)
                    ro = slice(SQ + r * BQ, SQ + (r + 1) * BQ)
                    qv = comm_ref[slot, rr, hc]
                    ks = kp_ref[r, :, :, hc].reshape(KV_R, DH)
                    vs = vp_ref[r, :, :, hc].reshape(KV_R, DH)
                    s = lax.dot_general(
                        qv, ks, (((1,), (1,)), ((), ())),
                        preferred_element_type=jnp.float32)
                    e = jnp.exp(s)
                    lsum = jnp.sum(e, axis=1, keepdims=True)
                    o = lax.dot_general(
                        e, vs, (((1,), (0,)), ((), ())),
                        preferred_element_type=jnp.float32)
                    if first:
                        comm_ref[slot, ro, hc] = o
                        l_ref[slot, rr, h:h + 1] = lsum
                    else:
                        comm_ref[slot, ro, hc] = comm_ref[slot, ro, hc] + o
                        l_ref[slot, rr, h:h + 1] = (
                            l_ref[slot, rr, h:h + 1] + lsum)

        comm_ref[0, 0:SQ, :] = q
        accumulate(0, first=True)

        for hop in range(N_DEV):
            rdma = pltpu.make_async_remote_copy(
                src_ref=comm_ref.at[hop],
                dst_ref=comm_ref.at[hop + 1],
                send_sem=send_sems.at[hop],
                recv_sem=recv_sems.at[hop],
                device_id=(right,),
                device_id_type=pl.DeviceIdType.MESH,
            )
            rdma_l = pltpu.make_async_remote_copy(
                src_ref=l_ref.at[hop],
                dst_ref=l_ref.at[hop + 1],
                send_sem=send_sems_l.at[hop],
                recv_sem=recv_sems_l.at[hop],
                device_id=(right,),
                device_id_type=pl.DeviceIdType.MESH,
            )
            rdma.start()
            rdma_l.start()
            rdma.wait()
            rdma_l.wait()
            if hop < N_DEV - 1:
                accumulate(hop + 1, first=False)

        rows = []
        for r in range(NB):
            blocks = []
            for h in range(HQ):
                hc = slice(h * DH, (h + 1) * DH)
                ro = slice(SQ + r * BQ, SQ + (r + 1) * BQ)
                rr = slice(r * BQ, (r + 1) * BQ)
                blocks.append(comm_ref[N_DEV, ro, hc]
                              / l_ref[N_DEV, rr, h:h + 1])
            rows.append(jnp.concatenate(blocks, axis=1))
        ctx = jnp.concatenate(rows, axis=0)
        out_ref[...] = jnp.dot(ctx, wo_ref[...],
                               preferred_element_type=jnp.float32)

    out = pl.pallas_call(
        body,
        out_shape=jax.ShapeDtypeStruct((SQ, D_MODEL), jnp.float32),
        in_specs=[
            pl.BlockSpec(memory_space=pltpu.VMEM),
            pl.BlockSpec(memory_space=pltpu.VMEM),
            pl.BlockSpec(memory_space=pl.ANY),
            pl.BlockSpec(memory_space=pl.ANY),
            pl.BlockSpec(memory_space=pltpu.VMEM),
        ],
        out_specs=pl.BlockSpec(memory_space=pltpu.VMEM),
        scratch_shapes=[
            pltpu.VMEM((N_DEV + 1, 2 * SQ, D_MODEL), jnp.float32),
            pltpu.VMEM((N_DEV + 1, SQ, HQ), jnp.float32),
            pltpu.VMEM((NB, GK, BQ, D_MODEL), jnp.float32),
            pltpu.VMEM((NB, GK, BQ, D_MODEL), jnp.float32),
            pltpu.SemaphoreType.DMA((N_DEV,)),
            pltpu.SemaphoreType.DMA((N_DEV,)),
            pltpu.SemaphoreType.DMA((N_DEV,)),
            pltpu.SemaphoreType.DMA((N_DEV,)),
            pltpu.SemaphoreType.DMA((2 * NB,)),
        ],
        compiler_params=pltpu.CompilerParams(
            collective_id=0, vmem_limit_bytes=60 * 1024 * 1024),
    )(x2, Wq, K4, V4, Wo)
    return out.reshape(1, SQ, D_MODEL)


# device time: 122484 ns/iter; 1.4765x vs baseline; 1.4765x over previous
import jax
import jax.numpy as jnp
from jax import lax
from jax.experimental import pallas as pl
from jax.experimental.pallas import tpu as pltpu

N_DEV = 4
SCALE = 0.08838834764831843
SQ = 256
SKV = 4096
HQ = 8
DH = 128
NB = 4
BQ = 64
GK = SKV // (NB * BQ)
KV_R = GK * BQ
D_MODEL = HQ * DH


def kernel(x, Wq, K_ext, V_ext, Wo):
    x2 = x.reshape(SQ, D_MODEL)
    K4 = K_ext.reshape(GK, NB, BQ, D_MODEL)
    V4 = V_ext.reshape(GK, NB, BQ, D_MODEL)

    def body(x_ref, wq_ref, k_ref, v_ref, wo_ref, out_ref,
             qbuf, olbuf, lbuf, pme_o, pme_l, ptmp_o, ptmp_l,
             kp_ref, vp_ref,
             qs_sems, qr_sems, os_sems, or_sems, ls_sems, lr_sems,
             kv_sems):
        my = lax.axis_index("i")
        left = (my + N_DEV - 1) % N_DEV
        right = (my + 1) % N_DEV

        kv_copies = []
        for r in range(NB):
            c = pltpu.make_async_copy(k_ref.at[:, r], kp_ref.at[r],
                                      kv_sems.at[r])
            c.start()
            kv_copies.append(c)
            c = pltpu.make_async_copy(v_ref.at[:, r], vp_ref.at[r],
                                      kv_sems.at[NB + r])
            c.start()
            kv_copies.append(c)

        barrier_sem = pltpu.get_barrier_semaphore()
        for nbr in (left, right):
            pl.semaphore_signal(barrier_sem, inc=1, device_id=(nbr,),
                                device_id_type=pl.DeviceIdType.MESH)
        pl.semaphore_wait(barrier_sem, 2)

        def ring(buf, ssems, rsems, k):
            return pltpu.make_async_remote_copy(
                src_ref=buf.at[k], dst_ref=buf.at[k + 1],
                send_sem=ssems.at[k], recv_sem=rsems.at[k],
                device_id=(right,), device_id_type=pl.DeviceIdType.MESH)

        qsend = [ring(qbuf, qs_sems, qr_sems, k) for k in range(N_DEV - 1)]
        osend = [ring(olbuf, os_sems, or_sems, k) for k in range(N_DEV - 1)]
        lsend = [ring(lbuf, ls_sems, lr_sems, k) for k in range(N_DEV - 1)]

        q = jnp.dot(x_ref[...], wq_ref[...],
                    preferred_element_type=jnp.float32) * SCALE
        qbuf[0, :, :] = q
        qsend[0].start()

        for c in kv_copies:
            c.wait()

        def partial(q_slot, o_dst, l_dst):
            for r in range(NB):
                rr = slice(r * BQ, (r + 1) * BQ)
                for h in range(HQ):
                    hc = slice(h * DH, (h + 1) * DH)
                    qv = qbuf[q_slot, rr, hc]
                    ks = kp_ref[r, :, :, hc].reshape(KV_R, DH)
                    vs = vp_ref[r, :, :, hc].reshape(KV_R, DH)
                    s = lax.dot_general(
                        qv, ks, (((1,), (1,)), ((), ())),
                        preferred_element_type=jnp.float32)
                    e = jnp.exp(s)
                    o_dst[rr, hc] = lax.dot_general(
                        e, vs, (((1,), (0,)), ((), ())),
                        preferred_element_type=jnp.float32)
                    l_dst[rr, h:h + 1] = jnp.sum(e, axis=1, keepdims=True)

        partial(0, pme_o, pme_l)

        qsend[0].wait_recv()
        qsend[1].start()
        partial(1, olbuf.at[0], lbuf.at[0])
        osend[0].start()
        lsend[0].start()

        qsend[1].wait_recv()
        qsend[2].start()
        partial(2, ptmp_o, ptmp_l)
        osend[0].wait_recv()
        lsend[0].wait_recv()
        olbuf[1, :, :] = olbuf[1, :, :] + ptmp_o[:, :]
        lbuf[1, :, :] = lbuf[1, :, :] + ptmp_l[:, :]
        osend[1].start()
        lsend[1].start()

        qsend[2].wait_recv()
        partial(3, ptmp_o, ptmp_l)
        osend[1].wait_recv()
        lsend[1].wait_recv()
        olbuf[2, :, :] = olbuf[2, :, :] + ptmp_o[:, :]
        lbuf[2, :, :] = lbuf[2, :, :] + ptmp_l[:, :]
        osend[2].start()
        lsend[2].start()

        osend[2].wait_recv()
        lsend[2].wait_recv()
        o_sum = olbuf[3, :, :] + pme_o[:, :]
        l_sum = lbuf[3, :, :] + pme_l[:, :]
        rows = []
        for r in range(NB):
            rr = slice(r * BQ, (r + 1) * BQ)
            blocks = [o_sum[rr, h * DH:(h + 1) * DH] / l_sum[rr, h:h + 1]
                      for h in range(HQ)]
            rows.append(jnp.concatenate(blocks, axis=1))
        ctx = jnp.concatenate(rows, axis=0)
        out_ref[...] = jnp.dot(ctx, wo_ref[...],
                               preferred_element_type=jnp.float32)

        for k in range(N_DEV - 1):
            qsend[k].wait_send()
            osend[k].wait_send()
            lsend[k].wait_send()

    out = pl.pallas_call(
        body,
        out_shape=jax.ShapeDtypeStruct((SQ, D_MODEL), jnp.float32),
        in_specs=[
            pl.BlockSpec(memory_space=pltpu.VMEM),
            pl.BlockSpec(memory_space=pltpu.VMEM),
            pl.BlockSpec(memory_space=pl.ANY),
            pl.BlockSpec(memory_space=pl.ANY),
            pl.BlockSpec(memory_space=pltpu.VMEM),
        ],
        out_specs=pl.BlockSpec(memory_space=pltpu.VMEM),
        scratch_shapes=[
            pltpu.VMEM((N_DEV, SQ, D_MODEL), jnp.float32),
            pltpu.VMEM((N_DEV, SQ, D_MODEL), jnp.float32),
            pltpu.VMEM((N_DEV, SQ, HQ), jnp.float32),
            pltpu.VMEM((SQ, D_MODEL), jnp.float32),
            pltpu.VMEM((SQ, HQ), jnp.float32),
            pltpu.VMEM((SQ, D_MODEL), jnp.float32),
            pltpu.VMEM((SQ, HQ), jnp.float32),
            pltpu.VMEM((NB, GK, BQ, D_MODEL), jnp.float32),
            pltpu.VMEM((NB, GK, BQ, D_MODEL), jnp.float32),
            pltpu.SemaphoreType.DMA((N_DEV - 1,)),
            pltpu.SemaphoreType.DMA((N_DEV - 1,)),
            pltpu.SemaphoreType.DMA((N_DEV - 1,)),
            pltpu.SemaphoreType.DMA((N_DEV - 1,)),
            pltpu.SemaphoreType.DMA((N_DEV - 1,)),
            pltpu.SemaphoreType.DMA((N_DEV - 1,)),
            pltpu.SemaphoreType.DMA((2 * NB,)),
        ],
        compiler_params=pltpu.CompilerParams(
            collective_id=0, vmem_limit_bytes=60 * 1024 * 1024),
    )(x2, Wq, K4, V4, Wo)
    return out.reshape(1, SQ, D_MODEL)


# device time: 104124 ns/iter; 1.7368x vs baseline; 1.1763x over previous
import jax
import jax.numpy as jnp
from jax import lax
from jax.experimental import pallas as pl
from jax.experimental.pallas import tpu as pltpu

N_DEV = 4
SCALE = 0.08838834764831843
SQ = 256
SKV = 4096
HQ = 8
DH = 128
NB = 4
BQ = 64
GK = SKV // (NB * BQ)
KV_R = GK * BQ
D_MODEL = HQ * DH


def kernel(x, Wq, K_ext, V_ext, Wo):
    x2 = x.reshape(SQ, D_MODEL)
    K4 = K_ext.reshape(GK, NB, BQ, D_MODEL).astype(jnp.bfloat16)
    V4 = V_ext.reshape(GK, NB, BQ, D_MODEL).astype(jnp.bfloat16)

    def body(x_ref, wq_ref, k_ref, v_ref, wo_ref, out_ref,
             qbuf, olbuf, lbuf, pme_o, pme_l, ptmp_o, ptmp_l,
             kp_ref, vp_ref,
             qs_sems, qr_sems, os_sems, or_sems, ls_sems, lr_sems,
             kv_sems):
        my = lax.axis_index("i")
        left = (my + N_DEV - 1) % N_DEV
        right = (my + 1) % N_DEV

        kv_copies = []
        for r in range(NB):
            c = pltpu.make_async_copy(k_ref.at[:, r], kp_ref.at[r],
                                      kv_sems.at[r])
            c.start()
            kv_copies.append(c)
            c = pltpu.make_async_copy(v_ref.at[:, r], vp_ref.at[r],
                                      kv_sems.at[NB + r])
            c.start()
            kv_copies.append(c)

        barrier_sem = pltpu.get_barrier_semaphore()
        for nbr in (left, right):
            pl.semaphore_signal(barrier_sem, inc=1, device_id=(nbr,),
                                device_id_type=pl.DeviceIdType.MESH)
        pl.semaphore_wait(barrier_sem, 2)

        def ring(buf, ssems, rsems, k):
            return pltpu.make_async_remote_copy(
                src_ref=buf.at[k], dst_ref=buf.at[k + 1],
                send_sem=ssems.at[k], recv_sem=rsems.at[k],
                device_id=(right,), device_id_type=pl.DeviceIdType.MESH)

        qsend = [ring(qbuf, qs_sems, qr_sems, k) for k in range(N_DEV - 1)]
        osend = [ring(olbuf, os_sems, or_sems, k) for k in range(N_DEV - 1)]
        lsend = [ring(lbuf, ls_sems, lr_sems, k) for k in range(N_DEV - 1)]

        q = jnp.dot(x_ref[...], wq_ref[...],
                    preferred_element_type=jnp.float32) * SCALE
        qbuf[0, :, :] = q.astype(jnp.bfloat16)
        qsend[0].start()

        for c in kv_copies:
            c.wait()

        def partial(q_slot, o_dst, l_dst):
            for r in range(NB):
                rr = slice(r * BQ, (r + 1) * BQ)
                for h in range(HQ):
                    hc = slice(h * DH, (h + 1) * DH)
                    qv = qbuf[q_slot, rr, hc]
                    ks = kp_ref[r, :, :, hc].reshape(KV_R, DH)
                    vs = vp_ref[r, :, :, hc].reshape(KV_R, DH)
                    s = lax.dot_general(
                        qv, ks, (((1,), (1,)), ((), ())),
                        preferred_element_type=jnp.float32)
                    e = jnp.exp(s)
                    o_dst[rr, hc] = lax.dot_general(
                        e.astype(jnp.bfloat16), vs, (((1,), (0,)), ((), ())),
                        preferred_element_type=jnp.float32)
                    l_dst[rr, h:h + 1] = jnp.sum(e, axis=1, keepdims=True)

        partial(0, pme_o, pme_l)

        qsend[0].wait_recv()
        qsend[1].start()
        partial(1, olbuf.at[0], lbuf.at[0])
        osend[0].start()
        lsend[0].start()

        qsend[1].wait_recv()
        qsend[2].start()
        partial(2, ptmp_o, ptmp_l)
        osend[0].wait_recv()
        lsend[0].wait_recv()
        olbuf[1, :, :] = olbuf[1, :, :] + ptmp_o[:, :]
        lbuf[1, :, :] = lbuf[1, :, :] + ptmp_l[:, :]
        osend[1].start()
        lsend[1].start()

        qsend[2].wait_recv()
        partial(3, ptmp_o, ptmp_l)
        osend[1].wait_recv()
        lsend[1].wait_recv()
        olbuf[2, :, :] = olbuf[2, :, :] + ptmp_o[:, :]
        lbuf[2, :, :] = lbuf[2, :, :] + ptmp_l[:, :]
        osend[2].start()
        lsend[2].start()

        osend[2].wait_recv()
        lsend[2].wait_recv()
        o_sum = olbuf[3, :, :] + pme_o[:, :]
        l_sum = lbuf[3, :, :] + pme_l[:, :]
        rows = []
        for r in range(NB):
            rr = slice(r * BQ, (r + 1) * BQ)
            blocks = [o_sum[rr, h * DH:(h + 1) * DH] / l_sum[rr, h:h + 1]
                      for h in range(HQ)]
            rows.append(jnp.concatenate(blocks, axis=1))
        ctx = jnp.concatenate(rows, axis=0)
        out_ref[...] = jnp.dot(ctx, wo_ref[...],
                               preferred_element_type=jnp.float32)

        for k in range(N_DEV - 1):
            qsend[k].wait_send()
            osend[k].wait_send()
            lsend[k].wait_send()

    out = pl.pallas_call(
        body,
        out_shape=jax.ShapeDtypeStruct((SQ, D_MODEL), jnp.float32),
        in_specs=[
            pl.BlockSpec(memory_space=pltpu.VMEM),
            pl.BlockSpec(memory_space=pltpu.VMEM),
            pl.BlockSpec(memory_space=pl.ANY),
            pl.BlockSpec(memory_space=pl.ANY),
            pl.BlockSpec(memory_space=pltpu.VMEM),
        ],
        out_specs=pl.BlockSpec(memory_space=pltpu.VMEM),
        scratch_shapes=[
            pltpu.VMEM((N_DEV, SQ, D_MODEL), jnp.bfloat16),
            pltpu.VMEM((N_DEV, SQ, D_MODEL), jnp.float32),
            pltpu.VMEM((N_DEV, SQ, HQ), jnp.float32),
            pltpu.VMEM((SQ, D_MODEL), jnp.float32),
            pltpu.VMEM((SQ, HQ), jnp.float32),
            pltpu.VMEM((SQ, D_MODEL), jnp.float32),
            pltpu.VMEM((SQ, HQ), jnp.float32),
            pltpu.VMEM((NB, GK, BQ, D_MODEL), jnp.bfloat16),
            pltpu.VMEM((NB, GK, BQ, D_MODEL), jnp.bfloat16),
            pltpu.SemaphoreType.DMA((N_DEV - 1,)),
            pltpu.SemaphoreType.DMA((N_DEV - 1,)),
            pltpu.SemaphoreType.DMA((N_DEV - 1,)),
            pltpu.SemaphoreType.DMA((N_DEV - 1,)),
            pltpu.SemaphoreType.DMA((N_DEV - 1,)),
            pltpu.SemaphoreType.DMA((N_DEV - 1,)),
            pltpu.SemaphoreType.DMA((2 * NB,)),
        ],
        compiler_params=pltpu.CompilerParams(
            collective_id=0, vmem_limit_bytes=60 * 1024 * 1024),
    )(x2, Wq, K4, V4, Wo)
    return out.reshape(1, SQ, D_MODEL)
